# baseline (speedup 1.0000x reference)
import functools

import jax
import jax.numpy as jnp
from jax import lax
from jax.experimental import pallas as pl
from jax.experimental.pallas import tpu as pltpu
from jax.experimental.pallas import tpu_sc as plsc


def kernel(sequence, table):
    batch = sequence.shape[0]
    seq_len = sequence.shape[2]
    vocab, dim = table.shape

    mesh = plsc.VectorSubcoreMesh(core_axis_name="c", subcore_axis_name="s")
    num_workers = mesh.num_cores * mesh.num_subcores

    assert seq_len % num_workers == 0
    rows_per_worker = seq_len // num_workers

    max_chunk = (131071 * 4) // (dim * table.dtype.itemsize) // 8 * 8
    chunks = []
    left = rows_per_worker
    while left > 0:
        c = min(max_chunk, left)
        chunks.append(c)
        left -= c
    buf_rows = chunks[0]

    @functools.partial(
        pl.kernel,
        out_type=jax.ShapeDtypeStruct((batch, seq_len, dim), table.dtype),
        mesh=mesh,
        scratch_types=[pltpu.VMEM((buf_rows, dim), table.dtype)],
    )
    def body(table_hbm, out_hbm, buf):
        wid = lax.axis_index("s") * mesh.num_cores + lax.axis_index("c")
        row0 = wid * rows_per_worker
        offset = 0
        for c in chunks:
            base = row0 + offset
            pltpu.sync_copy(table_hbm.at[pl.ds(base, c)], buf.at[pl.ds(0, c)])
            for b in range(batch):
                pltpu.sync_copy(buf.at[pl.ds(0, c)], out_hbm.at[b, pl.ds(base, c)])
            offset += c

    return body(table)
